# Initial kernel scaffold; baseline (speedup 1.0000x reference)
import jax
import jax.numpy as jnp
from jax.experimental import pallas as pl


def kernel(input_x, adj, W1, b1, W2, b2):
    raise NotImplementedError("write your pallas kernel here")


# speedup vs baseline: 57.4552x; 57.4552x over previous
import functools

import jax
import jax.numpy as jnp
from jax.experimental import pallas as pl


def _deg_kernel(adj_ref, out_ref, *, n_i):
    i = pl.program_id(1)
    s = jnp.sum(adj_ref[...], axis=0, keepdims=True)

    @pl.when(i == 0)
    def _init():
        out_ref[...] = s

    @pl.when(i > 0)
    def _acc():
        out_ref[...] += s

    @pl.when(i == n_i - 1)
    def _fin():
        out_ref[...] = jax.lax.rsqrt(out_ref[...] + 1.0)


def _dinv(adj, *, bi=512, bj=2048):
    n = adj.shape[0]
    n_i, n_j = n // bi, n // bj
    return pl.pallas_call(
        functools.partial(_deg_kernel, n_i=n_i),
        grid=(n_j, n_i),
        in_specs=[pl.BlockSpec((bi, bj), lambda j, i: (i, j))],
        out_specs=pl.BlockSpec((1, bj), lambda j, i: (0, j)),
        out_shape=jax.ShapeDtypeStruct((1, n), jnp.float32),
    )(adj)


def _xw_kernel(x_ref, w_ref, dinv_ref, out_ref):
    out_ref[...] = jnp.dot(
        x_ref[...], w_ref[...], preferred_element_type=jnp.float32
    ) * dinv_ref[...]


def _scaled_xw(x, w, dinv_col, *, br=1024):
    n, d_in = x.shape
    d_out = w.shape[1]
    return pl.pallas_call(
        _xw_kernel,
        grid=(n // br,),
        in_specs=[
            pl.BlockSpec((br, d_in), lambda r: (r, 0)),
            pl.BlockSpec((d_in, d_out), lambda r: (0, 0)),
            pl.BlockSpec((br, 1), lambda r: (r, 0)),
        ],
        out_specs=pl.BlockSpec((br, d_out), lambda r: (r, 0)),
        out_shape=jax.ShapeDtypeStruct((n, d_out), jnp.float32),
    )(x, w, dinv_col)


def _prop_kernel(adj_ref, y_ref, yj_ref, dinv_ref, b_ref, out_ref, *, n_i):
    i = pl.program_id(1)
    part = jax.lax.dot_general(
        adj_ref[...], y_ref[...],
        (((0,), (0,)), ((), ())),
        preferred_element_type=jnp.float32,
    )

    @pl.when(i == 0)
    def _init():
        out_ref[...] = part

    @pl.when(i > 0)
    def _acc():
        out_ref[...] += part

    @pl.when(i == n_i - 1)
    def _fin():
        acc = out_ref[...] + yj_ref[...]
        out_ref[...] = jnp.maximum(acc * dinv_ref[...] + b_ref[...], 0.0)


def _propagate(adj, y, dinv_col, b_row, *, bi=1024, bj=1024):
    n = adj.shape[0]
    d = y.shape[1]
    n_i, n_j = n // bi, n // bj
    return pl.pallas_call(
        functools.partial(_prop_kernel, n_i=n_i),
        grid=(n_j, n_i),
        in_specs=[
            pl.BlockSpec((bi, bj), lambda j, i: (i, j)),
            pl.BlockSpec((bi, d), lambda j, i: (i, 0)),
            pl.BlockSpec((bj, d), lambda j, i: (j, 0)),
            pl.BlockSpec((bj, 1), lambda j, i: (j, 0)),
            pl.BlockSpec((1, d), lambda j, i: (0, 0)),
        ],
        out_specs=pl.BlockSpec((bj, d), lambda j, i: (j, 0)),
        out_shape=jax.ShapeDtypeStruct((n, d), jnp.float32),
    )(adj, y, y, dinv_col, b_row)


def kernel(input_x, adj, W1, b1, W2, b2):
    n = adj.shape[0]
    x = input_x.astype(jnp.float32)
    dinv_row = _dinv(adj)
    dinv_col = dinv_row.reshape(n, 1)
    y1 = _scaled_xw(x, W1, dinv_col)
    h1 = _propagate(adj, y1, dinv_col, b1.reshape(1, -1))
    y2 = _scaled_xw(h1, W2, dinv_col)
    h2 = _propagate(adj, y2, dinv_col, b2.reshape(1, -1))
    return h2


# speedup vs baseline: 60.6751x; 1.0560x over previous
import functools

import jax
import jax.numpy as jnp
from jax.experimental import pallas as pl


def _deg_kernel(adj_ref, out_ref, cadj_ref, *, n_i):
    i = pl.program_id(1)
    a = adj_ref[...]
    cadj_ref[...] = a.astype(jnp.bfloat16)
    s = jnp.sum(a, axis=0, keepdims=True)

    @pl.when(i == 0)
    def _init():
        out_ref[...] = s

    @pl.when(i > 0)
    def _acc():
        out_ref[...] += s

    @pl.when(i == n_i - 1)
    def _fin():
        out_ref[...] = jax.lax.rsqrt(out_ref[...] + 1.0)


def _dinv_and_compress(adj, *, bi=512, bj=2048):
    n = adj.shape[0]
    n_i, n_j = n // bi, n // bj
    return pl.pallas_call(
        functools.partial(_deg_kernel, n_i=n_i),
        grid=(n_j, n_i),
        in_specs=[pl.BlockSpec((bi, bj), lambda j, i: (i, j))],
        out_specs=[
            pl.BlockSpec((1, bj), lambda j, i: (0, j)),
            pl.BlockSpec((bi, bj), lambda j, i: (i, j)),
        ],
        out_shape=[
            jax.ShapeDtypeStruct((1, n), jnp.float32),
            jax.ShapeDtypeStruct((n, n), jnp.bfloat16),
        ],
    )(adj)


def _xw_kernel(x_ref, w_ref, dinv_ref, out_ref):
    out_ref[...] = (jnp.dot(
        x_ref[...], w_ref[...], preferred_element_type=jnp.float32
    ) * dinv_ref[...]).astype(jnp.bfloat16)


def _scaled_xw(x, w, dinv_col, *, br=1024):
    n, d_in = x.shape
    d_out = w.shape[1]
    return pl.pallas_call(
        _xw_kernel,
        grid=(n // br,),
        in_specs=[
            pl.BlockSpec((br, d_in), lambda r: (r, 0)),
            pl.BlockSpec((d_in, d_out), lambda r: (0, 0)),
            pl.BlockSpec((br, 1), lambda r: (r, 0)),
        ],
        out_specs=pl.BlockSpec((br, d_out), lambda r: (r, 0)),
        out_shape=jax.ShapeDtypeStruct((n, d_out), jnp.bfloat16),
    )(x, w, dinv_col)


def _prop_kernel(adj_ref, y_ref, yj_ref, dinv_ref, b_ref, out_ref, *, n_i):
    i = pl.program_id(1)
    part = jax.lax.dot_general(
        adj_ref[...], y_ref[...],
        (((0,), (0,)), ((), ())),
        preferred_element_type=jnp.float32,
    )

    @pl.when(i == 0)
    def _init():
        out_ref[...] = part

    @pl.when(i > 0)
    def _acc():
        out_ref[...] += part

    @pl.when(i == n_i - 1)
    def _fin():
        acc = out_ref[...] + yj_ref[...].astype(jnp.float32)
        out_ref[...] = jnp.maximum(acc * dinv_ref[...] + b_ref[...], 0.0)


def _propagate(adj_c, y, dinv_col, b_row, *, bi=1024, bj=1024):
    n = adj_c.shape[0]
    d = y.shape[1]
    n_i, n_j = n // bi, n // bj
    return pl.pallas_call(
        functools.partial(_prop_kernel, n_i=n_i),
        grid=(n_j, n_i),
        in_specs=[
            pl.BlockSpec((bi, bj), lambda j, i: (i, j)),
            pl.BlockSpec((bi, d), lambda j, i: (i, 0)),
            pl.BlockSpec((bj, d), lambda j, i: (j, 0)),
            pl.BlockSpec((bj, 1), lambda j, i: (j, 0)),
            pl.BlockSpec((1, d), lambda j, i: (0, 0)),
        ],
        out_specs=pl.BlockSpec((bj, d), lambda j, i: (j, 0)),
        out_shape=jax.ShapeDtypeStruct((n, d), jnp.float32),
    )(adj_c, y, y, dinv_col, b_row)


def kernel(input_x, adj, W1, b1, W2, b2):
    n = adj.shape[0]
    x = input_x.astype(jnp.float32)
    dinv_row, adj_c = _dinv_and_compress(adj)
    dinv_col = dinv_row.reshape(n, 1)
    y1 = _scaled_xw(x, W1, dinv_col)
    h1 = _propagate(adj_c, y1, dinv_col, b1.reshape(1, -1))
    y2 = _scaled_xw(h1, W2, dinv_col)
    h2 = _propagate(adj_c, y2, dinv_col, b2.reshape(1, -1))
    return h2


# speedup vs baseline: 61.0818x; 1.0067x over previous
import functools

import jax
import jax.numpy as jnp
from jax.experimental import pallas as pl


def _deg_kernel(adj_ref, out_ref, cadj_ref, *, n_i):
    i = pl.program_id(1)
    a = adj_ref[...]
    cadj_ref[...] = a.T.astype(jnp.bfloat16)
    s = jnp.sum(a, axis=0, keepdims=True)

    @pl.when(i == 0)
    def _init():
        out_ref[...] = s

    @pl.when(i > 0)
    def _acc():
        out_ref[...] += s

    @pl.when(i == n_i - 1)
    def _fin():
        out_ref[...] = jax.lax.rsqrt(out_ref[...] + 1.0)


def _dinv_and_compress(adj, *, bi=512, bj=2048):
    n = adj.shape[0]
    n_i, n_j = n // bi, n // bj
    return pl.pallas_call(
        functools.partial(_deg_kernel, n_i=n_i),
        grid=(n_j, n_i),
        in_specs=[pl.BlockSpec((bi, bj), lambda j, i: (i, j))],
        out_specs=[
            pl.BlockSpec((1, bj), lambda j, i: (0, j)),
            pl.BlockSpec((bj, bi), lambda j, i: (j, i)),
        ],
        out_shape=[
            jax.ShapeDtypeStruct((1, n), jnp.float32),
            jax.ShapeDtypeStruct((n, n), jnp.bfloat16),
        ],
    )(adj)


def _xw_kernel(x_ref, w_ref, dinv_ref, out_ref):
    out_ref[...] = (jnp.dot(
        x_ref[...], w_ref[...], preferred_element_type=jnp.float32
    ) * dinv_ref[...]).astype(jnp.bfloat16)


def _scaled_xw(x, w, dinv_col, *, br=1024):
    n, d_in = x.shape
    d_out = w.shape[1]
    return pl.pallas_call(
        _xw_kernel,
        grid=(n // br,),
        in_specs=[
            pl.BlockSpec((br, d_in), lambda r: (r, 0)),
            pl.BlockSpec((d_in, d_out), lambda r: (0, 0)),
            pl.BlockSpec((br, 1), lambda r: (r, 0)),
        ],
        out_specs=pl.BlockSpec((br, d_out), lambda r: (r, 0)),
        out_shape=jax.ShapeDtypeStruct((n, d_out), jnp.bfloat16),
    )(x, w, dinv_col)


def _prop_kernel(adjt_ref, y_ref, yj_ref, dinv_ref, b_ref, out_ref, *, n_i):
    i = pl.program_id(1)
    part = jnp.dot(
        adjt_ref[...], y_ref[...], preferred_element_type=jnp.float32
    )

    @pl.when(i == 0)
    def _init():
        out_ref[...] = part

    @pl.when(i > 0)
    def _acc():
        out_ref[...] += part

    @pl.when(i == n_i - 1)
    def _fin():
        acc = out_ref[...] + yj_ref[...].astype(jnp.float32)
        out_ref[...] = jnp.maximum(acc * dinv_ref[...] + b_ref[...], 0.0)


def _propagate(adj_c, y, dinv_col, b_row, *, bi=1024, bj=1024):
    n = adj_c.shape[0]
    d = y.shape[1]
    n_i, n_j = n // bi, n // bj
    return pl.pallas_call(
        functools.partial(_prop_kernel, n_i=n_i),
        grid=(n_j, n_i),
        in_specs=[
            pl.BlockSpec((bj, bi), lambda j, i: (j, i)),
            pl.BlockSpec((bi, d), lambda j, i: (i, 0)),
            pl.BlockSpec((bj, d), lambda j, i: (j, 0)),
            pl.BlockSpec((bj, 1), lambda j, i: (j, 0)),
            pl.BlockSpec((1, d), lambda j, i: (0, 0)),
        ],
        out_specs=pl.BlockSpec((bj, d), lambda j, i: (j, 0)),
        out_shape=jax.ShapeDtypeStruct((n, d), jnp.float32),
    )(adj_c, y, y, dinv_col, b_row)


def kernel(input_x, adj, W1, b1, W2, b2):
    n = adj.shape[0]
    x = input_x.astype(jnp.float32)
    dinv_row, adj_c = _dinv_and_compress(adj)
    dinv_col = dinv_row.reshape(n, 1)
    y1 = _scaled_xw(x, W1, dinv_col)
    h1 = _propagate(adj_c, y1, dinv_col, b1.reshape(1, -1))
    y2 = _scaled_xw(h1, W2, dinv_col)
    h2 = _propagate(adj_c, y2, dinv_col, b2.reshape(1, -1))
    return h2


# speedup vs baseline: 69.6701x; 1.1406x over previous
import functools

import jax
import jax.numpy as jnp
from jax.experimental import pallas as pl


def _deg_kernel(adj_ref, out_ref, cadj_ref, *, n_i):
    i = pl.program_id(1)
    a = adj_ref[...]
    cadj_ref[...] = a.T.astype(jnp.bfloat16)
    s = jnp.sum(a, axis=0, keepdims=True)

    @pl.when(i == 0)
    def _init():
        out_ref[...] = s

    @pl.when(i > 0)
    def _acc():
        out_ref[...] += s

    @pl.when(i == n_i - 1)
    def _fin():
        out_ref[...] = jax.lax.rsqrt(out_ref[...] + 1.0)


def _dinv_and_compress(adj, *, bi=512, bj=2048):
    n = adj.shape[0]
    n_i, n_j = n // bi, n // bj
    return pl.pallas_call(
        functools.partial(_deg_kernel, n_i=n_i),
        grid=(n_j, n_i),
        in_specs=[pl.BlockSpec((bi, bj), lambda j, i: (i, j))],
        out_specs=[
            pl.BlockSpec((1, bj), lambda j, i: (0, j)),
            pl.BlockSpec((bj, bi), lambda j, i: (j, i)),
        ],
        out_shape=[
            jax.ShapeDtypeStruct((1, n), jnp.float32),
            jax.ShapeDtypeStruct((n, n), jnp.bfloat16),
        ],
    )(adj)


def _xw_kernel(x_ref, w_ref, dinv_ref, out_ref):
    out_ref[...] = (jnp.dot(
        x_ref[...], w_ref[...], preferred_element_type=jnp.float32
    ) * dinv_ref[...]).astype(jnp.bfloat16)


def _scaled_xw(x, w, dinv_col, *, br=1024):
    n, d_in = x.shape
    d_out = w.shape[1]
    return pl.pallas_call(
        _xw_kernel,
        grid=(n // br,),
        in_specs=[
            pl.BlockSpec((br, d_in), lambda r: (r, 0)),
            pl.BlockSpec((d_in, d_out), lambda r: (0, 0)),
            pl.BlockSpec((br, 1), lambda r: (r, 0)),
        ],
        out_specs=pl.BlockSpec((br, d_out), lambda r: (r, 0)),
        out_shape=jax.ShapeDtypeStruct((n, d_out), jnp.bfloat16),
    )(x, w, dinv_col)


def _prop_kernel(adjt_ref, y_ref, yj_ref, dinv_ref, b_ref, out_ref, *, n_i):
    i = pl.program_id(1)
    part = jnp.dot(
        adjt_ref[...], y_ref[...], preferred_element_type=jnp.float32
    )

    @pl.when(i == 0)
    def _init():
        out_ref[...] = part

    @pl.when(i > 0)
    def _acc():
        out_ref[...] += part

    @pl.when(i == n_i - 1)
    def _fin():
        acc = out_ref[...] + yj_ref[...].astype(jnp.float32)
        out_ref[...] = jnp.maximum(acc * dinv_ref[...] + b_ref[...], 0.0)


def _propagate(adj_c, y, dinv_col, b_row, *, bi=2048, bj=1024):
    n = adj_c.shape[0]
    d = y.shape[1]
    n_i, n_j = n // bi, n // bj
    return pl.pallas_call(
        functools.partial(_prop_kernel, n_i=n_i),
        grid=(n_j, n_i),
        in_specs=[
            pl.BlockSpec((bj, bi), lambda j, i: (j, i)),
            pl.BlockSpec((bi, d), lambda j, i: (i, 0)),
            pl.BlockSpec((bj, d), lambda j, i: (j, 0)),
            pl.BlockSpec((bj, 1), lambda j, i: (j, 0)),
            pl.BlockSpec((1, d), lambda j, i: (0, 0)),
        ],
        out_specs=pl.BlockSpec((bj, d), lambda j, i: (j, 0)),
        out_shape=jax.ShapeDtypeStruct((n, d), jnp.float32),
    )(adj_c, y, y, dinv_col, b_row)


def kernel(input_x, adj, W1, b1, W2, b2):
    n = adj.shape[0]
    x = input_x.astype(jnp.float32)
    dinv_row, adj_c = _dinv_and_compress(adj)
    dinv_col = dinv_row.reshape(n, 1)
    y1 = _scaled_xw(x, W1, dinv_col)
    h1 = _propagate(adj_c, y1, dinv_col, b1.reshape(1, -1))
    y2 = _scaled_xw(h1, W2, dinv_col)
    h2 = _propagate(adj_c, y2, dinv_col, b2.reshape(1, -1))
    return h2


# speedup vs baseline: 74.4669x; 1.0688x over previous
import functools

import jax
import jax.numpy as jnp
from jax.experimental import pallas as pl


def _deg_kernel(adj_ref, out_ref, cadj_ref, *, n_i):
    i = pl.program_id(1)
    a = adj_ref[...]
    cadj_ref[...] = a.T.astype(jnp.bfloat16)
    s = jnp.sum(a, axis=0, keepdims=True)

    @pl.when(i == 0)
    def _init():
        out_ref[...] = s

    @pl.when(i > 0)
    def _acc():
        out_ref[...] += s

    @pl.when(i == n_i - 1)
    def _fin():
        out_ref[...] = jax.lax.rsqrt(out_ref[...] + 1.0)


def _dinv_and_compress(adj, *, bi=512, bj=2048):
    n = adj.shape[0]
    n_i, n_j = n // bi, n // bj
    return pl.pallas_call(
        functools.partial(_deg_kernel, n_i=n_i),
        grid=(n_j, n_i),
        in_specs=[pl.BlockSpec((bi, bj), lambda j, i: (i, j))],
        out_specs=[
            pl.BlockSpec((1, bj), lambda j, i: (0, j)),
            pl.BlockSpec((bj, bi), lambda j, i: (j, i)),
        ],
        out_shape=[
            jax.ShapeDtypeStruct((1, n), jnp.float32),
            jax.ShapeDtypeStruct((n, n), jnp.bfloat16),
        ],
    )(adj)


def _xw_kernel(x_ref, w_ref, dinv_ref, out_ref):
    out_ref[...] = (jnp.dot(
        x_ref[...], w_ref[...], preferred_element_type=jnp.float32
    ) * dinv_ref[...]).astype(jnp.bfloat16)


def _scaled_xw(x, w, dinv_col, *, br=1024):
    n, d_in = x.shape
    d_out = w.shape[1]
    return pl.pallas_call(
        _xw_kernel,
        grid=(n // br,),
        in_specs=[
            pl.BlockSpec((br, d_in), lambda r: (r, 0)),
            pl.BlockSpec((d_in, d_out), lambda r: (0, 0)),
            pl.BlockSpec((br, 1), lambda r: (r, 0)),
        ],
        out_specs=pl.BlockSpec((br, d_out), lambda r: (r, 0)),
        out_shape=jax.ShapeDtypeStruct((n, d_out), jnp.bfloat16),
    )(x, w, dinv_col)


def _prop_kernel(adjt_ref, y_ref, yj_ref, dinv_ref, b_ref, out_ref, *, n_i):
    i = pl.program_id(1)
    part = jnp.dot(
        adjt_ref[...], y_ref[...], preferred_element_type=jnp.float32
    )

    @pl.when(i == 0)
    def _init():
        out_ref[...] = part

    @pl.when(i > 0)
    def _acc():
        out_ref[...] += part

    @pl.when(i == n_i - 1)
    def _fin():
        acc = out_ref[...] + yj_ref[...].astype(jnp.float32)
        out_ref[...] = jnp.maximum(acc * dinv_ref[...] + b_ref[...], 0.0)


def _propagate(adj_c, y, dinv_col, b_row, *, bi=4096, bj=1024):
    n = adj_c.shape[0]
    d = y.shape[1]
    n_i, n_j = n // bi, n // bj
    return pl.pallas_call(
        functools.partial(_prop_kernel, n_i=n_i),
        grid=(n_j, n_i),
        in_specs=[
            pl.BlockSpec((bj, bi), lambda j, i: (j, i)),
            pl.BlockSpec((bi, d), lambda j, i: (i, 0)),
            pl.BlockSpec((bj, d), lambda j, i: (j, 0)),
            pl.BlockSpec((bj, 1), lambda j, i: (j, 0)),
            pl.BlockSpec((1, d), lambda j, i: (0, 0)),
        ],
        out_specs=pl.BlockSpec((bj, d), lambda j, i: (j, 0)),
        out_shape=jax.ShapeDtypeStruct((n, d), jnp.float32),
    )(adj_c, y, y, dinv_col, b_row)


def kernel(input_x, adj, W1, b1, W2, b2):
    n = adj.shape[0]
    x = input_x.astype(jnp.float32)
    dinv_row, adj_c = _dinv_and_compress(adj)
    dinv_col = dinv_row.reshape(n, 1)
    y1 = _scaled_xw(x, W1, dinv_col)
    h1 = _propagate(adj_c, y1, dinv_col, b1.reshape(1, -1))
    y2 = _scaled_xw(h1, W2, dinv_col)
    h2 = _propagate(adj_c, y2, dinv_col, b2.reshape(1, -1))
    return h2


# speedup vs baseline: 75.0621x; 1.0080x over previous
import functools

import jax
import jax.numpy as jnp
from jax.experimental import pallas as pl


def _deg_kernel(adj_ref, out_ref, cadj_ref, *, n_i):
    i = pl.program_id(1)
    a = adj_ref[...]
    cadj_ref[...] = a.T.astype(jnp.bfloat16)
    s = jnp.sum(a, axis=0, keepdims=True)

    @pl.when(i == 0)
    def _init():
        out_ref[...] = s

    @pl.when(i > 0)
    def _acc():
        out_ref[...] += s

    @pl.when(i == n_i - 1)
    def _fin():
        out_ref[...] = jax.lax.rsqrt(out_ref[...] + 1.0)


def _dinv_and_compress(adj, *, bi=512, bj=2048):
    n = adj.shape[0]
    n_i, n_j = n // bi, n // bj
    return pl.pallas_call(
        functools.partial(_deg_kernel, n_i=n_i),
        grid=(n_j, n_i),
        in_specs=[pl.BlockSpec((bi, bj), lambda j, i: (i, j))],
        out_specs=[
            pl.BlockSpec((1, bj), lambda j, i: (0, j)),
            pl.BlockSpec((bj, bi), lambda j, i: (j, i)),
        ],
        out_shape=[
            jax.ShapeDtypeStruct((1, n), jnp.float32),
            jax.ShapeDtypeStruct((n, n), jnp.bfloat16),
        ],
    )(adj)


def _xw_kernel(x_ref, w_ref, dinv_ref, out_ref):
    out_ref[...] = (jnp.dot(
        x_ref[...], w_ref[...], preferred_element_type=jnp.float32
    ) * dinv_ref[...]).astype(jnp.bfloat16)


def _scaled_xw(x, w, dinv_col, *, br=1024):
    n, d_in = x.shape
    d_out = w.shape[1]
    return pl.pallas_call(
        _xw_kernel,
        grid=(n // br,),
        in_specs=[
            pl.BlockSpec((br, d_in), lambda r: (r, 0)),
            pl.BlockSpec((d_in, d_out), lambda r: (0, 0)),
            pl.BlockSpec((br, 1), lambda r: (r, 0)),
        ],
        out_specs=pl.BlockSpec((br, d_out), lambda r: (r, 0)),
        out_shape=jax.ShapeDtypeStruct((n, d_out), jnp.bfloat16),
    )(x, w, dinv_col)


def _prop_kernel(adjt_ref, y_ref, yj_ref, dinv_ref, b_ref, out_ref, *, n_i):
    i = pl.program_id(1)
    part = jnp.dot(
        adjt_ref[...], y_ref[...], preferred_element_type=jnp.float32
    )

    @pl.when(i == 0)
    def _init():
        out_ref[...] = part

    @pl.when(i > 0)
    def _acc():
        out_ref[...] += part

    @pl.when(i == n_i - 1)
    def _fin():
        acc = out_ref[...] + yj_ref[...].astype(jnp.float32)
        out_ref[...] = jnp.maximum(acc * dinv_ref[...] + b_ref[...], 0.0)


def _propagate(adj_c, y, dinv_col, b_row, *, bi=8192, bj=1024):
    n = adj_c.shape[0]
    d = y.shape[1]
    n_i, n_j = n // bi, n // bj
    return pl.pallas_call(
        functools.partial(_prop_kernel, n_i=n_i),
        grid=(n_j, n_i),
        in_specs=[
            pl.BlockSpec((bj, bi), lambda j, i: (j, i)),
            pl.BlockSpec((bi, d), lambda j, i: (i, 0)),
            pl.BlockSpec((bj, d), lambda j, i: (j, 0)),
            pl.BlockSpec((bj, 1), lambda j, i: (j, 0)),
            pl.BlockSpec((1, d), lambda j, i: (0, 0)),
        ],
        out_specs=pl.BlockSpec((bj, d), lambda j, i: (j, 0)),
        out_shape=jax.ShapeDtypeStruct((n, d), jnp.float32),
    )(adj_c, y, y, dinv_col, b_row)


def kernel(input_x, adj, W1, b1, W2, b2):
    n = adj.shape[0]
    x = input_x.astype(jnp.float32)
    dinv_row, adj_c = _dinv_and_compress(adj)
    dinv_col = dinv_row.reshape(n, 1)
    y1 = _scaled_xw(x, W1, dinv_col)
    h1 = _propagate(adj_c, y1, dinv_col, b1.reshape(1, -1))
    y2 = _scaled_xw(h1, W2, dinv_col)
    h2 = _propagate(adj_c, y2, dinv_col, b2.reshape(1, -1))
    return h2


# speedup vs baseline: 83.8572x; 1.1172x over previous
import functools

import jax
import jax.numpy as jnp
from jax.experimental import pallas as pl
from jax.experimental.pallas import tpu as pltpu


def _deg_kernel(adj_ref, dinv_ref, cadj_ref, *, n_i):
    i = pl.program_id(0)
    a = adj_ref[...]
    cadj_ref[...] = a.astype(jnp.bfloat16)
    s = jnp.sum(a, axis=0, keepdims=True)

    @pl.when(i == 0)
    def _init():
        dinv_ref[...] = s

    @pl.when(i > 0)
    def _acc():
        dinv_ref[...] += s

    @pl.when(i == n_i - 1)
    def _fin():
        dinv_ref[...] = jax.lax.rsqrt(dinv_ref[...] + 1.0)


def _dinv_and_compress(adj, *, bi=256):
    n = adj.shape[0]
    n_i = n // bi
    return pl.pallas_call(
        functools.partial(_deg_kernel, n_i=n_i),
        grid=(n_i,),
        in_specs=[pl.BlockSpec((bi, n), lambda i: (i, 0))],
        out_specs=[
            pl.BlockSpec((1, n), lambda i: (0, 0)),
            pl.BlockSpec((bi, n), lambda i: (i, 0)),
        ],
        out_shape=[
            jax.ShapeDtypeStruct((1, n), jnp.float32),
            jax.ShapeDtypeStruct((n, n), jnp.bfloat16),
        ],
    )(adj)


def _xw_t_kernel(x_ref, w_ref, dinv_ref, out_ref):
    y = jnp.dot(x_ref[...], w_ref[...], preferred_element_type=jnp.float32)
    out_ref[...] = (y * dinv_ref[...].reshape(-1, 1)).T.astype(jnp.bfloat16)


def _scaled_xw_t(x, w, dinv_row, *, br=2048):
    n, d_in = x.shape
    d_out = w.shape[1]
    return pl.pallas_call(
        _xw_t_kernel,
        grid=(n // br,),
        in_specs=[
            pl.BlockSpec((br, d_in), lambda r: (r, 0)),
            pl.BlockSpec((d_in, d_out), lambda r: (0, 0)),
            pl.BlockSpec((1, br), lambda r: (0, r)),
        ],
        out_specs=pl.BlockSpec((d_out, br), lambda r: (0, r)),
        out_shape=jax.ShapeDtypeStruct((d_out, n), jnp.bfloat16),
    )(x, w, dinv_row)


def _xw_t_from_t_kernel(h_t_ref, w_ref, dinv_ref, out_ref):
    y = jax.lax.dot_general(
        w_ref[...], h_t_ref[...],
        (((0,), (0,)), ((), ())),
        preferred_element_type=jnp.float32,
    )
    out_ref[...] = (y * dinv_ref[...]).astype(jnp.bfloat16)


def _scaled_xw_t_from_t(h_t, w, dinv_row):
    d, n = h_t.shape
    d_out = w.shape[1]
    return pl.pallas_call(
        _xw_t_from_t_kernel,
        grid=(1,),
        in_specs=[
            pl.BlockSpec((d, n), lambda r: (0, 0)),
            pl.BlockSpec((d, d_out), lambda r: (0, 0)),
            pl.BlockSpec((1, n), lambda r: (0, 0)),
        ],
        out_specs=pl.BlockSpec((d_out, n), lambda r: (0, 0)),
        out_shape=jax.ShapeDtypeStruct((d_out, n), jnp.bfloat16),
    )(h_t, w, dinv_row)


def _prop_kernel(adj_ref, yt_ref, ytfull_ref, dinv_ref, b_ref, out_ref,
                 acc_ref, *, n_i, transpose_out):
    i = pl.program_id(0)
    part = jnp.dot(
        yt_ref[...], adj_ref[...], preferred_element_type=jnp.float32
    )

    @pl.when(i == 0)
    def _init():
        acc_ref[...] = part + ytfull_ref[...].astype(jnp.float32)

    @pl.when(i > 0)
    def _acc():
        acc_ref[...] += part

    @pl.when(i == n_i - 1)
    def _fin():
        res = jnp.maximum(acc_ref[...] * dinv_ref[...] + b_ref[...], 0.0)
        if transpose_out:
            out_ref[...] = res.T
        else:
            out_ref[...] = res


def _propagate_t(adj_c, y_t, dinv_row, b_col, *, bi=512, transpose_out=False):
    n = adj_c.shape[0]
    d = y_t.shape[0]
    n_i = n // bi
    out_shape = (n, d) if transpose_out else (d, n)
    return pl.pallas_call(
        functools.partial(_prop_kernel, n_i=n_i, transpose_out=transpose_out),
        grid=(n_i,),
        in_specs=[
            pl.BlockSpec((bi, n), lambda i: (i, 0)),
            pl.BlockSpec((d, bi), lambda i: (0, i)),
            pl.BlockSpec((d, n), lambda i: (0, 0)),
            pl.BlockSpec((1, n), lambda i: (0, 0)),
            pl.BlockSpec((d, 1), lambda i: (0, 0)),
        ],
        out_specs=pl.BlockSpec(out_shape, lambda i: (0, 0)),
        out_shape=jax.ShapeDtypeStruct(out_shape, jnp.float32),
        scratch_shapes=[pltpu.VMEM((d, n), jnp.float32)],
    )(adj_c, y_t, y_t, dinv_row, b_col)


def kernel(input_x, adj, W1, b1, W2, b2):
    x = input_x.astype(jnp.float32)
    dinv_row, adj_c = _dinv_and_compress(adj)
    y1_t = _scaled_xw_t(x, W1, dinv_row)
    h1_t = _propagate_t(adj_c, y1_t, dinv_row, b1.reshape(-1, 1))
    y2_t = _scaled_xw_t_from_t(h1_t, W2, dinv_row)
    h2 = _propagate_t(adj_c, y2_t, dinv_row, b2.reshape(-1, 1),
                      transpose_out=True)
    return h2


# speedup vs baseline: 103.0655x; 1.2291x over previous
import functools

import jax
import jax.numpy as jnp
from jax.experimental import pallas as pl
from jax.experimental.pallas import tpu as pltpu


def _deg_kernel(adj_ref, dinv_ref, cadj_ref, *, n_i):
    i = pl.program_id(0)
    a = adj_ref[...]
    cadj_ref[...] = a.astype(jnp.int8)
    s = jnp.sum(a, axis=0, keepdims=True)

    @pl.when(i == 0)
    def _init():
        dinv_ref[...] = s

    @pl.when(i > 0)
    def _acc():
        dinv_ref[...] += s

    @pl.when(i == n_i - 1)
    def _fin():
        dinv_ref[...] = jax.lax.rsqrt(dinv_ref[...] + 1.0)


def _dinv_and_compress(adj, *, bi=256):
    n = adj.shape[0]
    n_i = n // bi
    return pl.pallas_call(
        functools.partial(_deg_kernel, n_i=n_i),
        grid=(n_i,),
        in_specs=[pl.BlockSpec((bi, n), lambda i: (i, 0))],
        out_specs=[
            pl.BlockSpec((1, n), lambda i: (0, 0)),
            pl.BlockSpec((bi, n), lambda i: (i, 0)),
        ],
        out_shape=[
            jax.ShapeDtypeStruct((1, n), jnp.float32),
            jax.ShapeDtypeStruct((n, n), jnp.int8),
        ],
    )(adj)


def _xw_t_kernel(x_ref, w_ref, dinv_ref, out_ref):
    y = jnp.dot(x_ref[...], w_ref[...], preferred_element_type=jnp.float32)
    out_ref[...] = (y * dinv_ref[...].reshape(-1, 1)).T.astype(jnp.bfloat16)


def _scaled_xw_t(x, w, dinv_row, *, br=2048):
    n, d_in = x.shape
    d_out = w.shape[1]
    return pl.pallas_call(
        _xw_t_kernel,
        grid=(n // br,),
        in_specs=[
            pl.BlockSpec((br, d_in), lambda r: (r, 0)),
            pl.BlockSpec((d_in, d_out), lambda r: (0, 0)),
            pl.BlockSpec((1, br), lambda r: (0, r)),
        ],
        out_specs=pl.BlockSpec((d_out, br), lambda r: (0, r)),
        out_shape=jax.ShapeDtypeStruct((d_out, n), jnp.bfloat16),
    )(x, w, dinv_row)


def _xw_t_from_t_kernel(h_t_ref, w_ref, dinv_ref, out_ref):
    y = jax.lax.dot_general(
        w_ref[...], h_t_ref[...],
        (((0,), (0,)), ((), ())),
        preferred_element_type=jnp.float32,
    )
    out_ref[...] = (y * dinv_ref[...]).astype(jnp.bfloat16)


def _scaled_xw_t_from_t(h_t, w, dinv_row):
    d, n = h_t.shape
    d_out = w.shape[1]
    return pl.pallas_call(
        _xw_t_from_t_kernel,
        grid=(1,),
        in_specs=[
            pl.BlockSpec((d, n), lambda r: (0, 0)),
            pl.BlockSpec((d, d_out), lambda r: (0, 0)),
            pl.BlockSpec((1, n), lambda r: (0, 0)),
        ],
        out_specs=pl.BlockSpec((d_out, n), lambda r: (0, 0)),
        out_shape=jax.ShapeDtypeStruct((d_out, n), jnp.bfloat16),
    )(h_t, w, dinv_row)


def _prop_kernel(adj_ref, yt_ref, ytfull_ref, dinv_ref, b_ref, out_ref,
                 acc_ref, *, n_i, transpose_out):
    i = pl.program_id(0)
    part = jnp.dot(
        yt_ref[...], adj_ref[...].astype(jnp.bfloat16),
        preferred_element_type=jnp.float32,
    )

    @pl.when(i == 0)
    def _init():
        acc_ref[...] = part + ytfull_ref[...].astype(jnp.float32)

    @pl.when(i > 0)
    def _acc():
        acc_ref[...] += part

    @pl.when(i == n_i - 1)
    def _fin():
        res = jnp.maximum(acc_ref[...] * dinv_ref[...] + b_ref[...], 0.0)
        if transpose_out:
            out_ref[...] = res.T
        else:
            out_ref[...] = res


def _propagate_t(adj_c, y_t, dinv_row, b_col, *, bi=512, transpose_out=False):
    n = adj_c.shape[0]
    d = y_t.shape[0]
    n_i = n // bi
    out_shape = (n, d) if transpose_out else (d, n)
    return pl.pallas_call(
        functools.partial(_prop_kernel, n_i=n_i, transpose_out=transpose_out),
        grid=(n_i,),
        in_specs=[
            pl.BlockSpec((bi, n), lambda i: (i, 0)),
            pl.BlockSpec((d, bi), lambda i: (0, i)),
            pl.BlockSpec((d, n), lambda i: (0, 0)),
            pl.BlockSpec((1, n), lambda i: (0, 0)),
            pl.BlockSpec((d, 1), lambda i: (0, 0)),
        ],
        out_specs=pl.BlockSpec(out_shape, lambda i: (0, 0)),
        out_shape=jax.ShapeDtypeStruct(out_shape, jnp.float32),
        scratch_shapes=[pltpu.VMEM((d, n), jnp.float32)],
    )(adj_c, y_t, y_t, dinv_row, b_col)


def kernel(input_x, adj, W1, b1, W2, b2):
    x = input_x.astype(jnp.float32)
    dinv_row, adj_c = _dinv_and_compress(adj)
    y1_t = _scaled_xw_t(x, W1, dinv_row)
    h1_t = _propagate_t(adj_c, y1_t, dinv_row, b1.reshape(-1, 1))
    y2_t = _scaled_xw_t_from_t(h1_t, W2, dinv_row)
    h2 = _propagate_t(adj_c, y2_t, dinv_row, b2.reshape(-1, 1),
                      transpose_out=True)
    return h2


# speedup vs baseline: 110.0272x; 1.0675x over previous
import functools

import jax
import jax.numpy as jnp
from jax.experimental import pallas as pl
from jax.experimental.pallas import tpu as pltpu


def _deg_kernel(adj_ref, dinv_ref, cadj_ref, *, n_i):
    i = pl.program_id(0)
    a = adj_ref[...]
    cadj_ref[...] = a.astype(jnp.int8)
    s = jnp.sum(a, axis=0, keepdims=True)

    @pl.when(i == 0)
    def _init():
        dinv_ref[...] = s

    @pl.when(i > 0)
    def _acc():
        dinv_ref[...] += s

    @pl.when(i == n_i - 1)
    def _fin():
        dinv_ref[...] = jax.lax.rsqrt(dinv_ref[...] + 1.0)


def _dinv_and_compress(adj, *, bi=256):
    n = adj.shape[0]
    n_i = n // bi
    return pl.pallas_call(
        functools.partial(_deg_kernel, n_i=n_i),
        grid=(n_i,),
        in_specs=[pl.BlockSpec((bi, n), lambda i: (i, 0))],
        out_specs=[
            pl.BlockSpec((1, n), lambda i: (0, 0)),
            pl.BlockSpec((bi, n), lambda i: (i, 0)),
        ],
        out_shape=[
            jax.ShapeDtypeStruct((1, n), jnp.float32),
            jax.ShapeDtypeStruct((n, n), jnp.int8),
        ],
    )(adj)


def _xw_t_kernel(x_ref, w_ref, dinv_ref, out_ref):
    y = jnp.dot(x_ref[...], w_ref[...], preferred_element_type=jnp.float32)
    out_ref[...] = (y * dinv_ref[...].reshape(-1, 1)).T.astype(jnp.bfloat16)


def _scaled_xw_t(x, w, dinv_row, *, br=2048):
    n, d_in = x.shape
    d_out = w.shape[1]
    return pl.pallas_call(
        _xw_t_kernel,
        grid=(n // br,),
        in_specs=[
            pl.BlockSpec((br, d_in), lambda r: (r, 0)),
            pl.BlockSpec((d_in, d_out), lambda r: (0, 0)),
            pl.BlockSpec((1, br), lambda r: (0, r)),
        ],
        out_specs=pl.BlockSpec((d_out, br), lambda r: (0, r)),
        out_shape=jax.ShapeDtypeStruct((d_out, n), jnp.bfloat16),
    )(x, w, dinv_row)


def _xw_t_from_t_kernel(h_t_ref, w_ref, dinv_ref, out_ref):
    y = jax.lax.dot_general(
        w_ref[...], h_t_ref[...],
        (((0,), (0,)), ((), ())),
        preferred_element_type=jnp.float32,
    )
    out_ref[...] = (y * dinv_ref[...]).astype(jnp.bfloat16)


def _scaled_xw_t_from_t(h_t, w, dinv_row):
    d, n = h_t.shape
    d_out = w.shape[1]
    return pl.pallas_call(
        _xw_t_from_t_kernel,
        grid=(1,),
        in_specs=[
            pl.BlockSpec((d, n), lambda r: (0, 0)),
            pl.BlockSpec((d, d_out), lambda r: (0, 0)),
            pl.BlockSpec((1, n), lambda r: (0, 0)),
        ],
        out_specs=pl.BlockSpec((d_out, n), lambda r: (0, 0)),
        out_shape=jax.ShapeDtypeStruct((d_out, n), jnp.bfloat16),
    )(h_t, w, dinv_row)


def _prop_kernel(adj_ref, yt_ref, ytp_ref, dinv_ref, b_ref, out_ref, *,
                 transpose_out):
    part = jnp.dot(
        yt_ref[...], adj_ref[...].astype(jnp.bfloat16),
        preferred_element_type=jnp.float32,
    )
    res = part + ytp_ref[...].astype(jnp.float32)
    res = jnp.maximum(res * dinv_ref[...] + b_ref[...], 0.0)
    if transpose_out:
        out_ref[...] = res.T
    else:
        out_ref[...] = res


def _propagate_t(adj_c, y_t, dinv_row, b_col, *, bj=1024, transpose_out=False):
    n = adj_c.shape[0]
    d = y_t.shape[0]
    n_j = n // bj
    out_shape = (n, d) if transpose_out else (d, n)
    out_block = (bj, d) if transpose_out else (d, bj)
    out_index = (lambda j: (j, 0)) if transpose_out else (lambda j: (0, j))
    return pl.pallas_call(
        functools.partial(_prop_kernel, transpose_out=transpose_out),
        grid=(n_j,),
        in_specs=[
            pl.BlockSpec((n, bj), lambda j: (0, j)),
            pl.BlockSpec((d, n), lambda j: (0, 0)),
            pl.BlockSpec((d, bj), lambda j: (0, j)),
            pl.BlockSpec((1, bj), lambda j: (0, j)),
            pl.BlockSpec((d, 1), lambda j: (0, 0)),
        ],
        out_specs=pl.BlockSpec(out_block, out_index),
        out_shape=jax.ShapeDtypeStruct(out_shape, jnp.float32),
    )(adj_c, y_t, y_t, dinv_row, b_col)


def kernel(input_x, adj, W1, b1, W2, b2):
    x = input_x.astype(jnp.float32)
    dinv_row, adj_c = _dinv_and_compress(adj)
    y1_t = _scaled_xw_t(x, W1, dinv_row)
    h1_t = _propagate_t(adj_c, y1_t, dinv_row, b1.reshape(-1, 1))
    y2_t = _scaled_xw_t_from_t(h1_t, W2, dinv_row)
    h2 = _propagate_t(adj_c, y2_t, dinv_row, b2.reshape(-1, 1),
                      transpose_out=True)
    return h2
